# baseline (speedup 1.0000x reference)
import functools

import jax
import jax.numpy as jnp
from jax import lax
from jax.experimental import pallas as pl
from jax.experimental.pallas import tpu as pltpu
from jax.experimental.pallas import tpu_sc as plsc

H = 256
H2 = 2 * H
SEQ_LEN = 10
B = 512
NUM_STEPS = B * SEQ_LEN
TOTAL_NEIGH = 81920
NUM_ENTS = 10000

L = 16
NC = 2
NS = 16
NW = NC * NS
HB = H // L

SPT = NUM_STEPS // NW
EPT = SPT // SEQ_LEN
CH = 16
CPD = 2
CHD = CH * CPD
NB_PAD = CHD

_MESH = plsc.VectorSubcoreMesh(core_axis_name="c", subcore_axis_name="s")

RPW = B // NW


@functools.partial(
    pl.kernel,
    out_type=(
        jax.ShapeDtypeStruct((B, H), jnp.float32),
        jax.ShapeDtypeStruct((B, H), jnp.float32),
    ),
    mesh=_MESH,
    scratch_types=[
        pltpu.VMEM((RPW,), jnp.int32),
        pltpu.VMEM((RPW,), jnp.int32),
        pltpu.VMEM((RPW, H), jnp.float32),
        pltpu.VMEM((RPW, H), jnp.float32),
        pltpu.SemaphoreType.DMA,
        pltpu.SemaphoreType.DMA,
    ],
)
def _sc_gather_sr(s_hbm, r_hbm, ee_hbm, re_hbm, es_out, rr_out,
                  sidx_v, ridx_v, esr_v, rrr_v, sem1, sem2):
    wid = lax.axis_index("s") * NC + lax.axis_index("c")
    base = wid * RPW
    pltpu.sync_copy(s_hbm.at[pl.ds(base, RPW)], sidx_v)
    pltpu.sync_copy(r_hbm.at[pl.ds(base, RPW)], ridx_v)
    c1 = pltpu.async_copy(ee_hbm.at[sidx_v], esr_v, sem1)
    c2 = pltpu.async_copy(re_hbm.at[ridx_v], rrr_v, sem2)
    c1.wait()
    c2.wait()
    pltpu.sync_copy(esr_v, es_out.at[pl.ds(base, RPW)])
    pltpu.sync_copy(rrr_v, rr_out.at[pl.ds(base, RPW)])


_RB = 1000


def _tc_proj_body(ee, w_attn, b2, es, rr, t_out, sr_out):
    i = pl.program_id(0)
    t_out[:, 0:H] = jnp.dot(ee[...], w_attn[0:H, :],
                            preferred_element_type=jnp.float32,
                            precision=lax.Precision.HIGHEST)
    t_out[:, H:H2] = ee[...]

    @pl.when(i == 0)
    def _():
        sr_out[...] = (
            jnp.dot(es[...], w_attn[H:2 * H, :],
                    preferred_element_type=jnp.float32,
                    precision=lax.Precision.HIGHEST)
            + jnp.dot(rr[...], w_attn[2 * H:3 * H, :],
                      preferred_element_type=jnp.float32,
                      precision=lax.Precision.HIGHEST)
            + b2[...])


_tc_proj = pl.pallas_call(
    _tc_proj_body,
    grid=(NUM_ENTS // _RB,),
    in_specs=[
        pl.BlockSpec((_RB, H), lambda i: (i, 0)),
        pl.BlockSpec((3 * H, H), lambda i: (0, 0)),
        pl.BlockSpec((1, H), lambda i: (0, 0)),
        pl.BlockSpec((B, H), lambda i: (0, 0)),
        pl.BlockSpec((B, H), lambda i: (0, 0)),
    ],
    out_specs=[
        pl.BlockSpec((_RB, H2), lambda i: (i, 0)),
        pl.BlockSpec((B, H), lambda i: (0, 0)),
    ],
    out_shape=[
        jax.ShapeDtypeStruct((NUM_ENTS, H2), jnp.float32),
        jax.ShapeDtypeStruct((B, H), jnp.float32),
    ],
)


@functools.partial(
    pl.kernel,
    out_type=jax.ShapeDtypeStruct((NUM_STEPS, H), jnp.float32),
    mesh=_MESH,
    scratch_types=[
        pltpu.VMEM((2, CHD), jnp.int32),
        pltpu.VMEM((2, CHD), jnp.int32),
        pltpu.VMEM((L,), jnp.int32),
        pltpu.VMEM((2, CHD, H2), jnp.float32),
        pltpu.VMEM((EPT, H), jnp.float32),
        pltpu.VMEM((SPT, H), jnp.float32),
        pltpu.VMEM((SPT, L), jnp.float32),
        pltpu.VMEM((H,), jnp.float32),
        pltpu.SemaphoreType.DMA,
        pltpu.SemaphoreType.DMA,
    ],
)
def _sc_main(nidx_hbm, seg_hbm, bnd_hbm, t_hbm, sr_hbm, vs_hbm, out_hbm,
             nidx_v, seg_v, bnd_v, trows, srloc, accn, dn, vs_v, semg, semi):
    c = lax.axis_index("c")
    sub = lax.axis_index("s")
    wid = sub * NC + c
    step_base = wid * SPT

    pltpu.sync_copy(vs_hbm, vs_v)
    pltpu.sync_copy(bnd_hbm.at[wid], bnd_v)
    pltpu.sync_copy(sr_hbm.at[pl.ds(pl.multiple_of(wid * EPT, 8), EPT)], srloc)

    def _zrow(i, _):
        for h in range(HB):
            accn[i, pl.ds(h * L, L)] = jnp.zeros((L,), jnp.float32)
        dn[i, pl.ds(0, L)] = jnp.zeros((L,), jnp.float32)
        return 0

    lax.fori_loop(0, SPT, _zrow, 0)

    rot_idx = [((lax.iota(jnp.int32, L) + k) & (L - 1)).reshape(L, 1)
               for k in (1, 2, 4, 8)]
    _dnums = lax.GatherDimensionNumbers(
        offset_dims=(), collapsed_slice_dims=(0,), start_index_map=(0,))

    def _all_lane_sum(v):
        for ri in rot_idx:
            v = v + lax.gather(v, ri, dimension_numbers=_dnums,
                               slice_sizes=(1,),
                               mode=lax.GatherScatterMode.PROMISE_IN_BOUNDS)
        return v

    vsum = vs_v[pl.ds(0, L)]
    for h in range(1, HB):
        vsum = vsum + vs_v[pl.ds(h * L, L)]
    vs_tot = _all_lane_sum(vsum)

    bv = bnd_v[pl.ds(0, L)]
    n_lo = bv[0]
    n_hi = bv[1]
    start = lax.shift_left(lax.shift_right_logical(n_lo, 3), 3)
    nchunks = lax.div(n_hi - start + (CHD - 1), CHD)

    nb0 = pl.multiple_of(start, 8)
    pltpu.sync_copy(nidx_hbm.at[pl.ds(nb0, CHD)], nidx_v.at[0])
    pltpu.sync_copy(seg_hbm.at[pl.ds(nb0, CHD)], seg_v.at[0])
    pltpu.async_copy(t_hbm.at[nidx_v.at[0]], trows.at[0], semg)

    def _chunk(g, _):
        slot = lax.rem(g, 2)
        nxt = 1 - slot
        nb = pl.multiple_of(start + g * CHD, 8)
        more = g + 1 < nchunks

        @pl.when(more)
        def _():
            nbn = pl.multiple_of(start + (g + 1) * CHD, 8)
            pltpu.async_copy(nidx_hbm.at[pl.ds(nbn, CHD)], nidx_v.at[nxt],
                             semi)
            pltpu.async_copy(seg_hbm.at[pl.ds(nbn, CHD)], seg_v.at[nxt], semi)

        pltpu.make_async_copy(t_hbm.at[nidx_v.at[slot]], trows.at[slot],
                              semg).wait()

        @pl.when(more)
        def _():
            nbn = pl.multiple_of(start + (g + 1) * CHD, 8)
            pltpu.make_async_copy(nidx_hbm.at[pl.ds(nbn, CHD)],
                                  nidx_v.at[nxt], semi).wait()
            pltpu.make_async_copy(seg_hbm.at[pl.ds(nbn, CHD)],
                                  seg_v.at[nxt], semi).wait()
            pltpu.async_copy(t_hbm.at[nidx_v.at[nxt]], trows.at[nxt], semg)

        def _blk(blk, _):
            segvec = seg_v[slot, pl.ds(blk * L, L)]
            rbase = blk * L

            for n in range(CH):
                pos = nb + rbase + n
                seg = segvec[n]
                srel = seg - step_base
                erow = lax.shift_right_logical(seg * 6554, 16) - wid * EPT

                valid = jnp.logical_and(pos >= n_lo, pos < n_hi)

                @pl.when(valid)
                def _():
                    acc = jnp.zeros((L,), jnp.float32)
                    for h in range(HB):
                        x = (trows[slot, rbase + n, pl.ds(h * L, L)]
                             + srloc[erow, pl.ds(h * L, L)])
                        acc = acc + vs_v[pl.ds(h * L, L)] / (jnp.exp(x + x)
                                                             + 1.0)
                    qtot = _all_lane_sum(acc)
                    w16 = jnp.exp(vs_tot - (qtot + qtot))
                    dn[srel, pl.ds(0, L)] = dn[srel, pl.ds(0, L)] + w16
                    for h in range(HB):
                        accn[srel, pl.ds(h * L, L)] = (
                            accn[srel, pl.ds(h * L, L)]
                            + w16 * trows[slot, rbase + n,
                                          pl.ds(H + h * L, L)])

            return 0

        lax.fori_loop(0, CPD, _blk, 0)
        return 0

    lax.fori_loop(0, nchunks, _chunk, 0)

    def _nrow(i, _):
        d = dn[i, pl.ds(0, L)]
        zero = jnp.zeros((L,), jnp.float32)
        for h in range(HB):
            v = accn[i, pl.ds(h * L, L)]
            accn[i, pl.ds(h * L, L)] = jnp.where(d > 0, v / d, zero)
        return 0

    lax.fori_loop(0, SPT, _nrow, 0)
    pltpu.sync_copy(accn, out_hbm.at[pl.ds(step_base, SPT)])


def kernel(s, r, neigh_idx, seg_ids, ent_embeds, rel_embeds, W_attn, b_attn, v_s):
    es, rr = _sc_gather_sr(s, r, ent_embeds, rel_embeds)
    t, sr = _tc_proj(ent_embeds, W_attn, b_attn.reshape(1, H), es, rr)
    bounds = jnp.searchsorted(
        seg_ids, jnp.arange(NW + 1, dtype=seg_ids.dtype) * SPT,
        side="left").astype(jnp.int32)
    bounds = jnp.concatenate(
        [bounds[:-1, None], bounds[1:, None],
         jnp.zeros((NW, L - 2), jnp.int32)], axis=1)
    nidx_p = jnp.pad(neigh_idx, (0, NB_PAD))
    seg_p = jnp.pad(seg_ids, (0, NB_PAD))
    return _sc_main(nidx_p, seg_p, bounds, t, sr, v_s.reshape(H))


# speedup vs baseline: 1.0626x; 1.0626x over previous
import functools

import jax
import jax.numpy as jnp
from jax import lax
from jax.experimental import pallas as pl
from jax.experimental.pallas import tpu as pltpu
from jax.experimental.pallas import tpu_sc as plsc

H = 256
H2 = 2 * H
SEQ_LEN = 10
B = 512
NUM_STEPS = B * SEQ_LEN
TOTAL_NEIGH = 81920
NUM_ENTS = 10000

L = 16
NC = 2
NS = 16
NW = NC * NS
HB = H // L

SPT = NUM_STEPS // NW
EPT = SPT // SEQ_LEN
CH = 16
NB_PAD = CH

_MESH = plsc.VectorSubcoreMesh(core_axis_name="c", subcore_axis_name="s")

RPW = B // NW


@functools.partial(
    pl.kernel,
    out_type=(
        jax.ShapeDtypeStruct((B, H), jnp.float32),
        jax.ShapeDtypeStruct((B, H), jnp.float32),
    ),
    mesh=_MESH,
    scratch_types=[
        pltpu.VMEM((RPW,), jnp.int32),
        pltpu.VMEM((RPW,), jnp.int32),
        pltpu.VMEM((RPW, H), jnp.float32),
        pltpu.VMEM((RPW, H), jnp.float32),
        pltpu.SemaphoreType.DMA,
        pltpu.SemaphoreType.DMA,
    ],
)
def _sc_gather_sr(s_hbm, r_hbm, ee_hbm, re_hbm, es_out, rr_out,
                  sidx_v, ridx_v, esr_v, rrr_v, sem1, sem2):
    wid = lax.axis_index("s") * NC + lax.axis_index("c")
    base = wid * RPW
    pltpu.sync_copy(s_hbm.at[pl.ds(base, RPW)], sidx_v)
    pltpu.sync_copy(r_hbm.at[pl.ds(base, RPW)], ridx_v)
    c1 = pltpu.async_copy(ee_hbm.at[sidx_v], esr_v, sem1)
    c2 = pltpu.async_copy(re_hbm.at[ridx_v], rrr_v, sem2)
    c1.wait()
    c2.wait()
    pltpu.sync_copy(esr_v, es_out.at[pl.ds(base, RPW)])
    pltpu.sync_copy(rrr_v, rr_out.at[pl.ds(base, RPW)])


_RB = 1000


def _tc_proj_body(ee, w_attn, b2, es, rr, t_out, sr_out):
    i = pl.program_id(0)
    t_out[:, 0:H] = jnp.dot(ee[...], w_attn[0:H, :],
                            preferred_element_type=jnp.float32,
                            precision=lax.Precision.HIGHEST)
    t_out[:, H:H2] = ee[...]

    @pl.when(i == 0)
    def _():
        sr_out[...] = (
            jnp.dot(es[...], w_attn[H:2 * H, :],
                    preferred_element_type=jnp.float32,
                    precision=lax.Precision.HIGHEST)
            + jnp.dot(rr[...], w_attn[2 * H:3 * H, :],
                      preferred_element_type=jnp.float32,
                      precision=lax.Precision.HIGHEST)
            + b2[...])


_tc_proj = pl.pallas_call(
    _tc_proj_body,
    grid=(NUM_ENTS // _RB,),
    in_specs=[
        pl.BlockSpec((_RB, H), lambda i: (i, 0)),
        pl.BlockSpec((3 * H, H), lambda i: (0, 0)),
        pl.BlockSpec((1, H), lambda i: (0, 0)),
        pl.BlockSpec((B, H), lambda i: (0, 0)),
        pl.BlockSpec((B, H), lambda i: (0, 0)),
    ],
    out_specs=[
        pl.BlockSpec((_RB, H2), lambda i: (i, 0)),
        pl.BlockSpec((B, H), lambda i: (0, 0)),
    ],
    out_shape=[
        jax.ShapeDtypeStruct((NUM_ENTS, H2), jnp.float32),
        jax.ShapeDtypeStruct((B, H), jnp.float32),
    ],
)


@functools.partial(
    pl.kernel,
    out_type=jax.ShapeDtypeStruct((NUM_STEPS, H), jnp.float32),
    mesh=_MESH,
    scratch_types=[
        pltpu.VMEM((2, CH), jnp.int32),
        pltpu.VMEM((2, CH), jnp.int32),
        pltpu.VMEM((L,), jnp.int32),
        pltpu.VMEM((2, CH, H2), jnp.float32),
        pltpu.VMEM((EPT, H), jnp.float32),
        pltpu.VMEM((SPT, H), jnp.float32),
        pltpu.VMEM((SPT, L), jnp.float32),
        pltpu.VMEM((H,), jnp.float32),
        pltpu.SemaphoreType.DMA,
        pltpu.SemaphoreType.DMA,
    ],
)
def _sc_main(nidx_hbm, seg_hbm, bnd_hbm, t_hbm, sr_hbm, vs_hbm, out_hbm,
             nidx_v, seg_v, bnd_v, trows, srloc, accn, dn, vs_v, semg, semi):
    c = lax.axis_index("c")
    sub = lax.axis_index("s")
    wid = sub * NC + c
    step_base = wid * SPT

    pltpu.sync_copy(vs_hbm, vs_v)
    pltpu.sync_copy(bnd_hbm.at[wid], bnd_v)
    pltpu.sync_copy(sr_hbm.at[pl.ds(pl.multiple_of(wid * EPT, 8), EPT)], srloc)

    def _zrow(i, _):
        for h in range(HB):
            accn[i, pl.ds(h * L, L)] = jnp.zeros((L,), jnp.float32)
        dn[i, pl.ds(0, L)] = jnp.zeros((L,), jnp.float32)
        return 0

    lax.fori_loop(0, SPT, _zrow, 0)

    rot_idx = [((lax.iota(jnp.int32, L) + k) & (L - 1)).reshape(L, 1)
               for k in (1, 2, 4, 8)]
    _dnums = lax.GatherDimensionNumbers(
        offset_dims=(), collapsed_slice_dims=(0,), start_index_map=(0,))

    def _all_lane_sum(v):
        for ri in rot_idx:
            v = v + lax.gather(v, ri, dimension_numbers=_dnums,
                               slice_sizes=(1,),
                               mode=lax.GatherScatterMode.PROMISE_IN_BOUNDS)
        return v

    vsum = vs_v[pl.ds(0, L)]
    for h in range(1, HB):
        vsum = vsum + vs_v[pl.ds(h * L, L)]
    vs_tot = _all_lane_sum(vsum)

    bv = bnd_v[pl.ds(0, L)]
    n_lo = bv[0]
    n_hi = bv[1]
    start = lax.shift_left(lax.shift_right_logical(n_lo, 3), 3)
    nchunks = lax.div(n_hi - start + (CH - 1), CH)

    nb0 = pl.multiple_of(start, 8)
    pltpu.sync_copy(nidx_hbm.at[pl.ds(nb0, CH)], nidx_v.at[0])
    pltpu.sync_copy(seg_hbm.at[pl.ds(nb0, CH)], seg_v.at[0])
    pltpu.async_copy(t_hbm.at[nidx_v.at[0]], trows.at[0], semg)

    def _chunk(g, _):
        slot = lax.rem(g, 2)
        nxt = 1 - slot
        nb = pl.multiple_of(start + g * CH, 8)
        more = g + 1 < nchunks

        @pl.when(more)
        def _():
            nbn = pl.multiple_of(start + (g + 1) * CH, 8)
            pltpu.async_copy(nidx_hbm.at[pl.ds(nbn, CH)], nidx_v.at[nxt], semi)
            pltpu.async_copy(seg_hbm.at[pl.ds(nbn, CH)], seg_v.at[nxt], semi)

        pltpu.make_async_copy(t_hbm.at[nidx_v.at[slot]], trows.at[slot],
                              semg).wait()

        @pl.when(more)
        def _():
            nbn = pl.multiple_of(start + (g + 1) * CH, 8)
            pltpu.make_async_copy(nidx_hbm.at[pl.ds(nbn, CH)],
                                  nidx_v.at[nxt], semi).wait()
            pltpu.make_async_copy(seg_hbm.at[pl.ds(nbn, CH)],
                                  seg_v.at[nxt], semi).wait()
            pltpu.async_copy(t_hbm.at[nidx_v.at[nxt]], trows.at[nxt], semg)

        segvec = seg_v[slot, pl.ds(0, L)]

        for n in range(CH):
            pos = nb + n
            seg = segvec[n]
            srel = seg - step_base
            erow = lax.shift_right_logical(seg * 6554, 16) - wid * EPT

            valid = jnp.logical_and(pos >= n_lo, pos < n_hi)

            @pl.when(valid)
            def _():
                accs = [jnp.zeros((L,), jnp.float32) for _ in range(4)]
                for h in range(HB):
                    x = (trows[slot, n, pl.ds(h * L, L)]
                         + srloc[erow, pl.ds(h * L, L)])
                    accs[h % 4] = (accs[h % 4]
                                   + vs_v[pl.ds(h * L, L)]
                                   / (jnp.exp(x + x) + 1.0))
                acc = (accs[0] + accs[1]) + (accs[2] + accs[3])
                qtot = _all_lane_sum(acc)
                w16 = jnp.exp(vs_tot - (qtot + qtot))
                dn[srel, pl.ds(0, L)] = dn[srel, pl.ds(0, L)] + w16
                for h in range(HB):
                    accn[srel, pl.ds(h * L, L)] = (
                        accn[srel, pl.ds(h * L, L)]
                        + w16 * trows[slot, n, pl.ds(H + h * L, L)])

        return 0

    lax.fori_loop(0, nchunks, _chunk, 0)

    def _nrow(i, _):
        d = dn[i, pl.ds(0, L)]
        zero = jnp.zeros((L,), jnp.float32)
        for h in range(HB):
            v = accn[i, pl.ds(h * L, L)]
            accn[i, pl.ds(h * L, L)] = jnp.where(d > 0, v / d, zero)
        return 0

    lax.fori_loop(0, SPT, _nrow, 0)
    pltpu.sync_copy(accn, out_hbm.at[pl.ds(step_base, SPT)])


def kernel(s, r, neigh_idx, seg_ids, ent_embeds, rel_embeds, W_attn, b_attn, v_s):
    es, rr = _sc_gather_sr(s, r, ent_embeds, rel_embeds)
    t, sr = _tc_proj(ent_embeds, W_attn, b_attn.reshape(1, H), es, rr)
    bounds = jnp.searchsorted(
        seg_ids, jnp.arange(NW + 1, dtype=seg_ids.dtype) * SPT,
        side="left").astype(jnp.int32)
    bounds = jnp.concatenate(
        [bounds[:-1, None], bounds[1:, None],
         jnp.zeros((NW, L - 2), jnp.int32)], axis=1)
    nidx_p = jnp.pad(neigh_idx, (0, NB_PAD))
    seg_p = jnp.pad(seg_ids, (0, NB_PAD))
    return _sc_main(nidx_p, seg_p, bounds, t, sr, v_s.reshape(H))
